# Initial kernel scaffold; baseline (speedup 1.0000x reference)
import jax
import jax.numpy as jnp
from jax.experimental import pallas as pl


def kernel(x, edge_index, edge_type, W1, root1, b1, W2, root2, b2):
    raise NotImplementedError("write your pallas kernel here")


# speedup vs baseline: 16.9436x; 16.9436x over previous
import functools

import jax
import jax.numpy as jnp
from jax import lax
from jax.experimental import pallas as pl
from jax.experimental.pallas import tpu as pltpu
from jax.experimental.pallas import tpu_sc as plsc

NC = 2
NS = 16
NW = NC * NS
L = 16

N = 10000
E = 320000
D = 128
R = 8

B0 = 80
EROWS = E // B0
NR = N * R
NPAD = 10240
TPW = NPAD // NS

_mesh = plsc.VectorSubcoreMesh(
    core_axis_name="c", subcore_axis_name="s", num_cores=NC, num_subcores=NS)


def _span(i, per, total, last):
    lo = pl.multiple_of(jnp.bitwise_and(i * per, -8), 8)
    hi = jnp.where(i == last, total, jnp.bitwise_and((i + 1) * per, -8))
    return lo, hi


def _al8(x):
    return pl.multiple_of(x, 8)


@functools.partial(
    pl.kernel,
    out_type=(
        jax.ShapeDtypeStruct((NR, L), jnp.float32),
        jax.ShapeDtypeStruct((EROWS, B0), jnp.int32),
        jax.ShapeDtypeStruct((EROWS, B0), jnp.int32),
    ),
    mesh=_mesh,
    compiler_params=pltpu.CompilerParams(use_tc_tiling_on_sc=False),
    scratch_types=(
        pltpu.VMEM_SHARED((NR, L), jnp.float32),
        pltpu.VMEM((1000, L), jnp.float32),
        pltpu.VMEM((B0, L), jnp.float32),
        pltpu.VMEM((8, B0), jnp.int32),
        pltpu.VMEM((8, B0), jnp.int32),
        pltpu.VMEM((8, B0), jnp.int32),
        pltpu.VMEM((8, B0), jnp.int32),
        pltpu.VMEM((8, B0), jnp.int32),
    ),
)
def _phase_p(src_h, dst_h, ty_h, wtab_h, sidx_h, cidx_h,
             cnt_sp, cbuf, ones_v, srcb, dstb, tyb, idxb, idxb2):
    cid = lax.axis_index("c")
    sid = lax.axis_index("s")
    zrow = jnp.zeros((L,), jnp.float32)
    onerow = jnp.ones((L,), jnp.float32)

    @pl.when(cid == 0)
    def _():
        def fill(r, _):
            cbuf[r] = zrow
            return 0
        lax.fori_loop(0, 1000, fill, 0)

        def zc(t, _):
            pltpu.sync_copy(cbuf, cnt_sp.at[pl.ds(_al8(sid * 5000 + t * 1000), 1000)])
            return 0
        lax.fori_loop(0, 5, zc, 0)

        def fill1(r, _):
            ones_v[r] = onerow
            return 0
        lax.fori_loop(0, B0, fill1, 0)

    plsc.subcore_barrier()

    @pl.when(cid == 0)
    def _():
        lo, hi = _span(sid, EROWS // NS, EROWS, NS - 1)

        def grp(g, _):
            base = _al8(lo + g * 8)
            pltpu.sync_copy(dst_h.at[pl.ds(base, 8)], dstb)
            pltpu.sync_copy(ty_h.at[pl.ds(base, 8)], tyb)

            def cmp(j, _):
                for k in range(B0 // L):
                    d = dstb[j, pl.ds(k * L, L)]
                    t = tyb[j, pl.ds(k * L, L)]
                    idxb[j, pl.ds(k * L, L)] = d * R + t
                return 0
            lax.fori_loop(0, 8, cmp, 0)

            def sc(j, _):
                pltpu.sync_copy(ones_v, cnt_sp.at[idxb.at[j]], add=True)
                return 0
            lax.fori_loop(0, 8, sc, 0)
            return 0
        lax.fori_loop(0, lax.shift_right_arithmetic(hi - lo, 3), grp, 0)

    @pl.when(cid == 1)
    def _():
        lo, hi = _span(sid, EROWS // NS, EROWS, NS - 1)

        def grp(g, _):
            base = _al8(lo + g * 8)
            pltpu.sync_copy(src_h.at[pl.ds(base, 8)], srcb)
            pltpu.sync_copy(dst_h.at[pl.ds(base, 8)], dstb)
            pltpu.sync_copy(ty_h.at[pl.ds(base, 8)], tyb)

            def cmp(j, _):
                for k in range(B0 // L):
                    s = srcb[j, pl.ds(k * L, L)]
                    d = dstb[j, pl.ds(k * L, L)]
                    t = tyb[j, pl.ds(k * L, L)]
                    idxb[j, pl.ds(k * L, L)] = s * R + t
                    idxb2[j, pl.ds(k * L, L)] = d * R + t
                return 0
            lax.fori_loop(0, 8, cmp, 0)
            pltpu.sync_copy(idxb, sidx_h.at[pl.ds(base, 8)])
            pltpu.sync_copy(idxb2, cidx_h.at[pl.ds(base, 8)])
            return 0
        lax.fori_loop(0, lax.shift_right_arithmetic(hi - lo, 3), grp, 0)

    plsc.subcore_barrier()

    @pl.when(cid == 0)
    def _():
        def tc(t, _):
            base = _al8(sid * 5000 + t * 1000)
            pltpu.sync_copy(cnt_sp.at[pl.ds(base, 1000)], cbuf)

            def inv(r, _):
                cbuf[r] = 1.0 / jnp.maximum(cbuf[r], 1.0)
                return 0
            lax.fori_loop(0, 1000, inv, 0)
            pltpu.sync_copy(cbuf, wtab_h.at[pl.ds(base, 1000)])
            return 0
        lax.fori_loop(0, 5, tc, 0)


@functools.partial(
    pl.kernel,
    out_type=jax.ShapeDtypeStruct((NC * NPAD, D), jnp.float32),
    mesh=_mesh,
    compiler_params=pltpu.CompilerParams(use_tc_tiling_on_sc=False),
    scratch_types=(
        pltpu.VMEM_SHARED((NPAD, D), jnp.float32),
        pltpu.VMEM((B0, D), jnp.float32),
        pltpu.VMEM((B0, L), jnp.float32),
        pltpu.VMEM((8, B0), jnp.int32),
        pltpu.VMEM((8, B0), jnp.int32),
        pltpu.VMEM((8, B0), jnp.int32),
        pltpu.VMEM((128, D), jnp.float32),
        pltpu.SemaphoreType.DMA,
        pltpu.SemaphoreType.DMA,
    ),
)
def _layer_pass(y_h, sidx_h, dst_h, cidx_h, wtab_h, parts_h,
                acc_sp, rows_v, w_v, sidxb, dstb, cidxb, stage, sem_g, sem_w):
    cid = lax.axis_index("c")
    sid = lax.axis_index("s")
    wid = cid * NS + sid
    z16 = jnp.zeros((L,), jnp.float32)

    def fillz(i, _):
        for k in range(D // L):
            stage[i, pl.ds(k * L, L)] = z16
        return 0
    lax.fori_loop(0, 128, fillz, 0)

    def zc(t, _):
        pltpu.sync_copy(stage, acc_sp.at[pl.ds(_al8(sid * TPW + t * 128), 128)])
        return 0
    lax.fori_loop(0, TPW // 128, zc, 0)

    plsc.subcore_barrier()

    lo, hi = _span(wid, EROWS // NW, EROWS, NW - 1)

    def grp(g, _):
        base = _al8(lo + g * 8)
        pltpu.sync_copy(sidx_h.at[pl.ds(base, 8)], sidxb)
        pltpu.sync_copy(dst_h.at[pl.ds(base, 8)], dstb)
        pltpu.sync_copy(cidx_h.at[pl.ds(base, 8)], cidxb)

        def one(j, _):
            dg = pltpu.async_copy(y_h.at[sidxb.at[j]], rows_v, sem_g)
            dw = pltpu.async_copy(wtab_h.at[cidxb.at[j]], w_v, sem_w)
            dg.wait()
            dw.wait()

            def scale(e, _):
                wv = w_v[e]
                for k in range(D // L):
                    rows_v[e, pl.ds(k * L, L)] = rows_v[e, pl.ds(k * L, L)] * wv
                return 0
            lax.fori_loop(0, B0, scale, 0)
            pltpu.sync_copy(rows_v, acc_sp.at[dstb.at[j]], add=True)
            return 0
        lax.fori_loop(0, 8, one, 0)
        return 0
    lax.fori_loop(0, lax.shift_right_arithmetic(hi - lo, 3), grp, 0)

    plsc.subcore_barrier()

    def wb(t, _):
        o = _al8(sid * TPW + t * 128)
        pltpu.sync_copy(acc_sp.at[pl.ds(o, 128)], stage)
        pltpu.sync_copy(stage, parts_h.at[pl.ds(_al8(cid * NPAD + o), 128)])
        return 0
    lax.fori_loop(0, TPW // 128, wb, 0)


BN = 400


def _tc_a_body(x_ref, wcat_ref, root_ref, b_ref, y_ref, base_ref):
    xb = x_ref[...]
    y_ref[...] = jnp.dot(xb, wcat_ref[...], preferred_element_type=jnp.float32)
    base_ref[...] = (jnp.dot(xb, root_ref[...], preferred_element_type=jnp.float32)
                     + b_ref[...])


def _tc_b_body(base1_ref, parts_ref, wcat_ref, root_ref, b_ref, y_ref, base2_ref):
    h = jnp.maximum(base1_ref[...] + parts_ref[0] + parts_ref[1], 0.0)
    y_ref[...] = jnp.dot(h, wcat_ref[...], preferred_element_type=jnp.float32)
    base2_ref[...] = (jnp.dot(h, root_ref[...], preferred_element_type=jnp.float32)
                      + b_ref[...])


def _tc_c_body(base2_ref, parts_ref, out_ref):
    out_ref[...] = base2_ref[...] + parts_ref[0] + parts_ref[1]


def _tc_a(x, wcat, root, b):
    return pl.pallas_call(
        _tc_a_body,
        grid=(N // BN,),
        in_specs=[
            pl.BlockSpec((BN, D), lambda i: (i, 0)),
            pl.BlockSpec((D, R * D), lambda i: (0, 0)),
            pl.BlockSpec((D, D), lambda i: (0, 0)),
            pl.BlockSpec((1, D), lambda i: (0, 0)),
        ],
        out_specs=[
            pl.BlockSpec((BN, R * D), lambda i: (i, 0)),
            pl.BlockSpec((BN, D), lambda i: (i, 0)),
        ],
        out_shape=[
            jax.ShapeDtypeStruct((N, R * D), jnp.float32),
            jax.ShapeDtypeStruct((N, D), jnp.float32),
        ],
    )(x, wcat, root, b)


def _tc_b(base1, parts, wcat, root, b):
    return pl.pallas_call(
        _tc_b_body,
        grid=(N // BN,),
        in_specs=[
            pl.BlockSpec((BN, D), lambda i: (i, 0)),
            pl.BlockSpec((NC, BN, D), lambda i: (0, i, 0)),
            pl.BlockSpec((D, R * D), lambda i: (0, 0)),
            pl.BlockSpec((D, D), lambda i: (0, 0)),
            pl.BlockSpec((1, D), lambda i: (0, 0)),
        ],
        out_specs=[
            pl.BlockSpec((BN, R * D), lambda i: (i, 0)),
            pl.BlockSpec((BN, D), lambda i: (i, 0)),
        ],
        out_shape=[
            jax.ShapeDtypeStruct((N, R * D), jnp.float32),
            jax.ShapeDtypeStruct((N, D), jnp.float32),
        ],
    )(base1, parts, wcat, root, b)


def _tc_c(base2, parts):
    return pl.pallas_call(
        _tc_c_body,
        grid=(N // BN,),
        in_specs=[
            pl.BlockSpec((BN, D), lambda i: (i, 0)),
            pl.BlockSpec((NC, BN, D), lambda i: (0, i, 0)),
        ],
        out_specs=pl.BlockSpec((BN, D), lambda i: (i, 0)),
        out_shape=jax.ShapeDtypeStruct((N, D), jnp.float32),
    )(base2, parts)


def kernel(x, edge_index, edge_type, W1, root1, b1, W2, root2, b2):
    src2 = edge_index[0].reshape(EROWS, B0)
    dst2 = edge_index[1].reshape(EROWS, B0)
    ty2 = edge_type.reshape(EROWS, B0)
    wcat1 = W1.transpose(1, 0, 2).reshape(D, R * D)
    wcat2 = W2.transpose(1, 0, 2).reshape(D, R * D)
    b1r = b1.reshape(1, D)
    b2r = b2.reshape(1, D)

    wtab, sidx2, cidx2 = _phase_p(src2, dst2, ty2)

    y1, base1 = _tc_a(x, wcat1, root1, b1r)
    parts1 = _layer_pass(y1.reshape(N * R, D), sidx2, dst2, cidx2, wtab)
    y2, base2 = _tc_b(base1, parts1.reshape(NC, NPAD, D), wcat2, root2, b2r)
    parts2 = _layer_pass(y2.reshape(N * R, D), sidx2, dst2, cidx2, wtab)
    return _tc_c(base2, parts2.reshape(NC, NPAD, D))


# speedup vs baseline: 23.4558x; 1.3843x over previous
import functools

import jax
import jax.numpy as jnp
from jax import lax
from jax.experimental import pallas as pl
from jax.experimental.pallas import tpu as pltpu
from jax.experimental.pallas import tpu_sc as plsc

NC = 2
NS = 16
NW = NC * NS
L = 16

N = 10000
E = 320000
D = 128
R = 8

B0 = 80
EROWS = E // B0
NR = N * R
NPAD = 10240
TPW = NPAD // NS

_mesh = plsc.VectorSubcoreMesh(
    core_axis_name="c", subcore_axis_name="s", num_cores=NC, num_subcores=NS)


def _span(i, per, total, last):
    lo = pl.multiple_of(jnp.bitwise_and(i * per, -8), 8)
    hi = jnp.where(i == last, total, jnp.bitwise_and((i + 1) * per, -8))
    return lo, hi


def _al8(x):
    return pl.multiple_of(x, 8)


@functools.partial(
    pl.kernel,
    out_type=(
        jax.ShapeDtypeStruct((NR, L), jnp.float32),
        jax.ShapeDtypeStruct((EROWS, B0), jnp.int32),
        jax.ShapeDtypeStruct((EROWS, B0), jnp.int32),
    ),
    mesh=_mesh,
    compiler_params=pltpu.CompilerParams(use_tc_tiling_on_sc=False),
    scratch_types=(
        pltpu.VMEM_SHARED((NR, L), jnp.float32),
        pltpu.VMEM((1000, L), jnp.float32),
        pltpu.VMEM((B0, L), jnp.float32),
        pltpu.VMEM((8, B0), jnp.int32),
        pltpu.VMEM((8, B0), jnp.int32),
        pltpu.VMEM((8, B0), jnp.int32),
        pltpu.VMEM((8, B0), jnp.int32),
        pltpu.VMEM((8, B0), jnp.int32),
    ),
)
def _phase_p(src_h, dst_h, ty_h, wtab_h, sidx_h, cidx_h,
             cnt_sp, cbuf, ones_v, srcb, dstb, tyb, idxb, idxb2):
    cid = lax.axis_index("c")
    sid = lax.axis_index("s")
    zrow = jnp.zeros((L,), jnp.float32)
    onerow = jnp.ones((L,), jnp.float32)

    @pl.when(cid == 0)
    def _():
        def fill(r, _):
            cbuf[r] = zrow
            return 0
        lax.fori_loop(0, 1000, fill, 0)

        def zc(t, _):
            pltpu.sync_copy(cbuf, cnt_sp.at[pl.ds(_al8(sid * 5000 + t * 1000), 1000)])
            return 0
        lax.fori_loop(0, 5, zc, 0)

        def fill1(r, _):
            ones_v[r] = onerow
            return 0
        lax.fori_loop(0, B0, fill1, 0)

    plsc.subcore_barrier()

    @pl.when(cid == 0)
    def _():
        lo, hi = _span(sid, EROWS // NS, EROWS, NS - 1)

        def grp(g, _):
            base = _al8(lo + g * 8)
            pltpu.sync_copy(dst_h.at[pl.ds(base, 8)], dstb)
            pltpu.sync_copy(ty_h.at[pl.ds(base, 8)], tyb)

            def cmp(j, _):
                for k in range(B0 // L):
                    d = dstb[j, pl.ds(k * L, L)]
                    t = tyb[j, pl.ds(k * L, L)]
                    idxb[j, pl.ds(k * L, L)] = d * R + t
                return 0
            lax.fori_loop(0, 8, cmp, 0)

            def sc(j, _):
                pltpu.sync_copy(ones_v, cnt_sp.at[idxb.at[j]], add=True)
                return 0
            lax.fori_loop(0, 8, sc, 0)
            return 0
        lax.fori_loop(0, lax.shift_right_arithmetic(hi - lo, 3), grp, 0)

    @pl.when(cid == 1)
    def _():
        lo, hi = _span(sid, EROWS // NS, EROWS, NS - 1)

        def grp(g, _):
            base = _al8(lo + g * 8)
            pltpu.sync_copy(src_h.at[pl.ds(base, 8)], srcb)
            pltpu.sync_copy(dst_h.at[pl.ds(base, 8)], dstb)
            pltpu.sync_copy(ty_h.at[pl.ds(base, 8)], tyb)

            def cmp(j, _):
                for k in range(B0 // L):
                    s = srcb[j, pl.ds(k * L, L)]
                    d = dstb[j, pl.ds(k * L, L)]
                    t = tyb[j, pl.ds(k * L, L)]
                    idxb[j, pl.ds(k * L, L)] = s * R + t
                    idxb2[j, pl.ds(k * L, L)] = d * R + t
                return 0
            lax.fori_loop(0, 8, cmp, 0)
            pltpu.sync_copy(idxb, sidx_h.at[pl.ds(base, 8)])
            pltpu.sync_copy(idxb2, cidx_h.at[pl.ds(base, 8)])
            return 0
        lax.fori_loop(0, lax.shift_right_arithmetic(hi - lo, 3), grp, 0)

    plsc.subcore_barrier()

    @pl.when(cid == 0)
    def _():
        def tc(t, _):
            base = _al8(sid * 5000 + t * 1000)
            pltpu.sync_copy(cnt_sp.at[pl.ds(base, 1000)], cbuf)

            def inv(r, _):
                cbuf[r] = 1.0 / jnp.maximum(cbuf[r], 1.0)
                return 0
            lax.fori_loop(0, 1000, inv, 0)
            pltpu.sync_copy(cbuf, wtab_h.at[pl.ds(base, 1000)])
            return 0
        lax.fori_loop(0, 5, tc, 0)


@functools.partial(
    pl.kernel,
    out_type=jax.ShapeDtypeStruct((NC * NPAD, D), jnp.float32),
    mesh=_mesh,
    compiler_params=pltpu.CompilerParams(use_tc_tiling_on_sc=False),
    scratch_types=(
        pltpu.VMEM_SHARED((NPAD, D), jnp.float32),
        pltpu.VMEM((B0, D), jnp.float32),
        pltpu.VMEM((B0, D), jnp.float32),
        pltpu.VMEM((B0, L), jnp.float32),
        pltpu.VMEM((B0, L), jnp.float32),
        pltpu.VMEM((8, B0), jnp.int32),
        pltpu.VMEM((8, B0), jnp.int32),
        pltpu.VMEM((8, B0), jnp.int32),
        pltpu.VMEM((128, D), jnp.float32),
        pltpu.SemaphoreType.DMA,
        pltpu.SemaphoreType.DMA,
        pltpu.SemaphoreType.DMA,
        pltpu.SemaphoreType.DMA,
    ),
)
def _layer_pass(y_h, sidx_h, dst_h, cidx_h, wtab_h, parts_h,
                acc_sp, rows_a, rows_b, w_a, w_b, sidxb, dstb, cidxb, stage,
                sem_ga, sem_gb, sem_wa, sem_wb):
    cid = lax.axis_index("c")
    sid = lax.axis_index("s")
    wid = cid * NS + sid
    z16 = jnp.zeros((L,), jnp.float32)

    def fillz(i, _):
        for k in range(D // L):
            stage[i, pl.ds(k * L, L)] = z16
        return 0
    lax.fori_loop(0, 128, fillz, 0)

    def zc(t, _):
        pltpu.sync_copy(stage, acc_sp.at[pl.ds(_al8(sid * TPW + t * 128), 128)])
        return 0
    lax.fori_loop(0, TPW // 128, zc, 0)

    plsc.subcore_barrier()

    lo, hi = _span(wid, EROWS // NW, EROWS, NW - 1)

    def start(j, rows_v, w_v, sem_g, sem_w):
        pltpu.async_copy(y_h.at[sidxb.at[j]], rows_v, sem_g)
        pltpu.async_copy(wtab_h.at[cidxb.at[j]], w_v, sem_w)

    def finish(j, rows_v, w_v, sem_g, sem_w):
        pltpu.make_async_copy(y_h.at[sidxb.at[j]], rows_v, sem_g).wait()
        pltpu.make_async_copy(wtab_h.at[cidxb.at[j]], w_v, sem_w).wait()

        def scale(p, _):
            e = 2 * p
            wv0 = w_v[e]
            wv1 = w_v[e + 1]
            for k in range(D // L):
                rows_v[e, pl.ds(k * L, L)] = rows_v[e, pl.ds(k * L, L)] * wv0
                rows_v[e + 1, pl.ds(k * L, L)] = (
                    rows_v[e + 1, pl.ds(k * L, L)] * wv1)
            return 0
        lax.fori_loop(0, B0 // 2, scale, 0)
        pltpu.sync_copy(rows_v, acc_sp.at[dstb.at[j]], add=True)

    def grp(g, _):
        base = _al8(lo + g * 8)
        pltpu.sync_copy(sidx_h.at[pl.ds(base, 8)], sidxb)
        pltpu.sync_copy(dst_h.at[pl.ds(base, 8)], dstb)
        pltpu.sync_copy(cidx_h.at[pl.ds(base, 8)], cidxb)

        start(0, rows_a, w_a, sem_ga, sem_wa)

        def pair(p, _):
            j = 2 * p
            start(j + 1, rows_b, w_b, sem_gb, sem_wb)
            finish(j, rows_a, w_a, sem_ga, sem_wa)

            @pl.when(p < 3)
            def _():
                start(j + 2, rows_a, w_a, sem_ga, sem_wa)
            finish(j + 1, rows_b, w_b, sem_gb, sem_wb)
            return 0
        lax.fori_loop(0, 4, pair, 0)
        return 0
    lax.fori_loop(0, lax.shift_right_arithmetic(hi - lo, 3), grp, 0)

    plsc.subcore_barrier()

    def wb(t, _):
        o = _al8(sid * TPW + t * 128)
        pltpu.sync_copy(acc_sp.at[pl.ds(o, 128)], stage)
        pltpu.sync_copy(stage, parts_h.at[pl.ds(_al8(cid * NPAD + o), 128)])
        return 0
    lax.fori_loop(0, TPW // 128, wb, 0)


BN = 400


def _tc_a_body(x_ref, wcat_ref, root_ref, b_ref, y_ref, base_ref):
    xb = x_ref[...]
    y_ref[...] = jnp.dot(xb, wcat_ref[...], preferred_element_type=jnp.float32)
    base_ref[...] = (jnp.dot(xb, root_ref[...], preferred_element_type=jnp.float32)
                     + b_ref[...])


def _tc_b_body(base1_ref, parts_ref, wcat_ref, root_ref, b_ref, y_ref, base2_ref):
    h = jnp.maximum(base1_ref[...] + parts_ref[0] + parts_ref[1], 0.0)
    y_ref[...] = jnp.dot(h, wcat_ref[...], preferred_element_type=jnp.float32)
    base2_ref[...] = (jnp.dot(h, root_ref[...], preferred_element_type=jnp.float32)
                      + b_ref[...])


def _tc_c_body(base2_ref, parts_ref, out_ref):
    out_ref[...] = base2_ref[...] + parts_ref[0] + parts_ref[1]


def _tc_a(x, wcat, root, b):
    return pl.pallas_call(
        _tc_a_body,
        grid=(N // BN,),
        in_specs=[
            pl.BlockSpec((BN, D), lambda i: (i, 0)),
            pl.BlockSpec((D, R * D), lambda i: (0, 0)),
            pl.BlockSpec((D, D), lambda i: (0, 0)),
            pl.BlockSpec((1, D), lambda i: (0, 0)),
        ],
        out_specs=[
            pl.BlockSpec((BN, R * D), lambda i: (i, 0)),
            pl.BlockSpec((BN, D), lambda i: (i, 0)),
        ],
        out_shape=[
            jax.ShapeDtypeStruct((N, R * D), jnp.float32),
            jax.ShapeDtypeStruct((N, D), jnp.float32),
        ],
    )(x, wcat, root, b)


def _tc_b(base1, parts, wcat, root, b):
    return pl.pallas_call(
        _tc_b_body,
        grid=(N // BN,),
        in_specs=[
            pl.BlockSpec((BN, D), lambda i: (i, 0)),
            pl.BlockSpec((NC, BN, D), lambda i: (0, i, 0)),
            pl.BlockSpec((D, R * D), lambda i: (0, 0)),
            pl.BlockSpec((D, D), lambda i: (0, 0)),
            pl.BlockSpec((1, D), lambda i: (0, 0)),
        ],
        out_specs=[
            pl.BlockSpec((BN, R * D), lambda i: (i, 0)),
            pl.BlockSpec((BN, D), lambda i: (i, 0)),
        ],
        out_shape=[
            jax.ShapeDtypeStruct((N, R * D), jnp.float32),
            jax.ShapeDtypeStruct((N, D), jnp.float32),
        ],
    )(base1, parts, wcat, root, b)


def _tc_c(base2, parts):
    return pl.pallas_call(
        _tc_c_body,
        grid=(N // BN,),
        in_specs=[
            pl.BlockSpec((BN, D), lambda i: (i, 0)),
            pl.BlockSpec((NC, BN, D), lambda i: (0, i, 0)),
        ],
        out_specs=pl.BlockSpec((BN, D), lambda i: (i, 0)),
        out_shape=jax.ShapeDtypeStruct((N, D), jnp.float32),
    )(base2, parts)


def kernel(x, edge_index, edge_type, W1, root1, b1, W2, root2, b2):
    src2 = edge_index[0].reshape(EROWS, B0)
    dst2 = edge_index[1].reshape(EROWS, B0)
    ty2 = edge_type.reshape(EROWS, B0)
    wcat1 = W1.transpose(1, 0, 2).reshape(D, R * D)
    wcat2 = W2.transpose(1, 0, 2).reshape(D, R * D)
    b1r = b1.reshape(1, D)
    b2r = b2.reshape(1, D)

    wtab, sidx2, cidx2 = _phase_p(src2, dst2, ty2)

    y1, base1 = _tc_a(x, wcat1, root1, b1r)
    parts1 = _layer_pass(y1.reshape(N * R, D), sidx2, dst2, cidx2, wtab)
    y2, base2 = _tc_b(base1, parts1.reshape(NC, NPAD, D), wcat2, root2, b2r)
    parts2 = _layer_pass(y2.reshape(N * R, D), sidx2, dst2, cidx2, wtab)
    return _tc_c(base2, parts2.reshape(NC, NPAD, D))


# speedup vs baseline: 26.4003x; 1.1255x over previous
import functools

import jax
import jax.numpy as jnp
from jax import lax
from jax.experimental import pallas as pl
from jax.experimental.pallas import tpu as pltpu
from jax.experimental.pallas import tpu_sc as plsc

NC = 2
NS = 16
NW = NC * NS
L = 16

N = 10000
E = 320000
D = 128
R = 8

B0 = 80
EROWS = E // B0
NR = N * R
NPAD = 10240
TPW = NPAD // NS

_mesh = plsc.VectorSubcoreMesh(
    core_axis_name="c", subcore_axis_name="s", num_cores=NC, num_subcores=NS)


def _span(i, per, total, last):
    lo = pl.multiple_of(jnp.bitwise_and(i * per, -8), 8)
    hi = jnp.where(i == last, total, jnp.bitwise_and((i + 1) * per, -8))
    return lo, hi


def _al8(x):
    return pl.multiple_of(x, 8)


@functools.partial(
    pl.kernel,
    out_type=(
        jax.ShapeDtypeStruct((NR, L), jnp.float32),
        jax.ShapeDtypeStruct((EROWS, B0), jnp.int32),
        jax.ShapeDtypeStruct((EROWS, B0), jnp.int32),
    ),
    mesh=_mesh,
    compiler_params=pltpu.CompilerParams(use_tc_tiling_on_sc=False),
    scratch_types=(
        pltpu.VMEM_SHARED((NR, L), jnp.float32),
        pltpu.VMEM((1000, L), jnp.float32),
        pltpu.VMEM((B0, L), jnp.float32),
        pltpu.VMEM((8, B0), jnp.int32),
        pltpu.VMEM((8, B0), jnp.int32),
        pltpu.VMEM((8, B0), jnp.int32),
        pltpu.VMEM((8, B0), jnp.int32),
        pltpu.VMEM((8, B0), jnp.int32),
    ),
)
def _phase_p(src_h, dst_h, ty_h, wtab_h, sidx_h, cidx_h,
             cnt_sp, cbuf, ones_v, srcb, dstb, tyb, idxb, idxb2):
    cid = lax.axis_index("c")
    sid = lax.axis_index("s")
    zrow = jnp.zeros((L,), jnp.float32)
    onerow = jnp.ones((L,), jnp.float32)

    @pl.when(cid == 0)
    def _():
        def fill(r, _):
            cbuf[r] = zrow
            return 0
        lax.fori_loop(0, 1000, fill, 0)

        def zc(t, _):
            pltpu.sync_copy(cbuf, cnt_sp.at[pl.ds(_al8(sid * 5000 + t * 1000), 1000)])
            return 0
        lax.fori_loop(0, 5, zc, 0)

        def fill1(r, _):
            ones_v[r] = onerow
            return 0
        lax.fori_loop(0, B0, fill1, 0)

    plsc.subcore_barrier()

    @pl.when(cid == 0)
    def _():
        lo, hi = _span(sid, EROWS // NS, EROWS, NS - 1)

        def grp(g, _):
            base = _al8(lo + g * 8)
            pltpu.sync_copy(dst_h.at[pl.ds(base, 8)], dstb)
            pltpu.sync_copy(ty_h.at[pl.ds(base, 8)], tyb)

            def cmp(j, _):
                for k in range(B0 // L):
                    d = dstb[j, pl.ds(k * L, L)]
                    t = tyb[j, pl.ds(k * L, L)]
                    idxb[j, pl.ds(k * L, L)] = d * R + t
                return 0
            lax.fori_loop(0, 8, cmp, 0)

            def sc(j, _):
                pltpu.sync_copy(ones_v, cnt_sp.at[idxb.at[j]], add=True)
                return 0
            lax.fori_loop(0, 8, sc, 0)
            return 0
        lax.fori_loop(0, lax.shift_right_arithmetic(hi - lo, 3), grp, 0)

    @pl.when(cid == 1)
    def _():
        lo, hi = _span(sid, EROWS // NS, EROWS, NS - 1)

        def grp(g, _):
            base = _al8(lo + g * 8)
            pltpu.sync_copy(src_h.at[pl.ds(base, 8)], srcb)
            pltpu.sync_copy(dst_h.at[pl.ds(base, 8)], dstb)
            pltpu.sync_copy(ty_h.at[pl.ds(base, 8)], tyb)

            def cmp(j, _):
                for k in range(B0 // L):
                    s = srcb[j, pl.ds(k * L, L)]
                    d = dstb[j, pl.ds(k * L, L)]
                    t = tyb[j, pl.ds(k * L, L)]
                    idxb[j, pl.ds(k * L, L)] = s * R + t
                    idxb2[j, pl.ds(k * L, L)] = d * R + t
                return 0
            lax.fori_loop(0, 8, cmp, 0)
            pltpu.sync_copy(idxb, sidx_h.at[pl.ds(base, 8)])
            pltpu.sync_copy(idxb2, cidx_h.at[pl.ds(base, 8)])
            return 0
        lax.fori_loop(0, lax.shift_right_arithmetic(hi - lo, 3), grp, 0)

    plsc.subcore_barrier()

    @pl.when(cid == 0)
    def _():
        def tc(t, _):
            base = _al8(sid * 5000 + t * 1000)
            pltpu.sync_copy(cnt_sp.at[pl.ds(base, 1000)], cbuf)

            def inv(r, _):
                cbuf[r] = 1.0 / jnp.maximum(cbuf[r], 1.0)
                return 0
            lax.fori_loop(0, 1000, inv, 0)
            pltpu.sync_copy(cbuf, wtab_h.at[pl.ds(base, 1000)])
            return 0
        lax.fori_loop(0, 5, tc, 0)


@functools.partial(
    pl.kernel,
    out_type=jax.ShapeDtypeStruct((NC * NPAD, D), jnp.float32),
    mesh=_mesh,
    compiler_params=pltpu.CompilerParams(use_tc_tiling_on_sc=False),
    scratch_types=(
        pltpu.VMEM_SHARED((NPAD, D), jnp.float32),
        pltpu.VMEM((B0, D), jnp.float32),
        pltpu.VMEM((B0, D), jnp.float32),
        pltpu.VMEM((B0, L), jnp.float32),
        pltpu.VMEM((B0, L), jnp.float32),
        pltpu.VMEM((8, B0), jnp.int32),
        pltpu.VMEM((8, B0), jnp.int32),
        pltpu.VMEM((8, B0), jnp.int32),
        pltpu.VMEM((128, D), jnp.float32),
        pltpu.SemaphoreType.DMA,
        pltpu.SemaphoreType.DMA,
        pltpu.SemaphoreType.DMA,
        pltpu.SemaphoreType.DMA,
    ),
)
def _layer_pass(y_h, sidx_h, dst_h, cidx_h, wtab_h, parts_h,
                acc_sp, rows_a, rows_b, w_a, w_b, sidxb, dstb, cidxb, stage,
                sem_ga, sem_gb, sem_wa, sem_wb):
    cid = lax.axis_index("c")
    sid = lax.axis_index("s")
    wid = cid * NS + sid
    z16 = jnp.zeros((L,), jnp.float32)

    def fillz(i, _):
        for k in range(D // L):
            stage[i, pl.ds(k * L, L)] = z16
        return 0
    lax.fori_loop(0, 128, fillz, 0)

    def zc(t, _):
        pltpu.sync_copy(stage, acc_sp.at[pl.ds(_al8(sid * TPW + t * 128), 128)])
        return 0
    lax.fori_loop(0, TPW // 128, zc, 0)

    plsc.subcore_barrier()

    lo, hi = _span(wid, EROWS // NW, EROWS, NW - 1)

    def start(j, rows_v, w_v, sem_g, sem_w):
        pltpu.async_copy(y_h.at[sidxb.at[j]], rows_v, sem_g)
        pltpu.async_copy(wtab_h.at[cidxb.at[j]], w_v, sem_w)

    def finish(j, rows_v, w_v, sem_g, sem_w):
        pltpu.make_async_copy(y_h.at[sidxb.at[j]], rows_v, sem_g).wait()
        pltpu.make_async_copy(wtab_h.at[cidxb.at[j]], w_v, sem_w).wait()

        def scale(p, _):
            e = 2 * p
            wv0 = w_v[e]
            wv1 = w_v[e + 1]
            for k in range(D // L):
                rows_v[e, pl.ds(k * L, L)] = rows_v[e, pl.ds(k * L, L)] * wv0
                rows_v[e + 1, pl.ds(k * L, L)] = (
                    rows_v[e + 1, pl.ds(k * L, L)] * wv1)
            return 0
        lax.fori_loop(0, B0 // 2, scale, 0)
        pltpu.sync_copy(rows_v, acc_sp.at[dstb.at[j]], add=True)

    def grp(g, _):
        base = _al8(lo + g * 8)
        pltpu.sync_copy(sidx_h.at[pl.ds(base, 8)], sidxb)
        pltpu.sync_copy(dst_h.at[pl.ds(base, 8)], dstb)
        pltpu.sync_copy(cidx_h.at[pl.ds(base, 8)], cidxb)

        start(0, rows_a, w_a, sem_ga, sem_wa)

        def pair(p, _):
            j = 2 * p
            start(j + 1, rows_b, w_b, sem_gb, sem_wb)
            finish(j, rows_a, w_a, sem_ga, sem_wa)

            @pl.when(p < 3)
            def _():
                start(j + 2, rows_a, w_a, sem_ga, sem_wa)
            finish(j + 1, rows_b, w_b, sem_gb, sem_wb)
            return 0
        lax.fori_loop(0, 4, pair, 0)
        return 0
    lax.fori_loop(0, lax.shift_right_arithmetic(hi - lo, 3), grp, 0)

    plsc.subcore_barrier()

    def wb(t, _):
        o = _al8(sid * TPW + t * 128)
        pltpu.sync_copy(acc_sp.at[pl.ds(o, 128)], stage)
        pltpu.sync_copy(stage, parts_h.at[pl.ds(_al8(cid * NPAD + o), 128)])
        return 0
    lax.fori_loop(0, TPW // 128, wb, 0)


BN = 400


def _tc_a_body(x_ref, wcat_ref, root_ref, b_ref, y_ref, base_ref):
    xb = x_ref[...]
    y_ref[...] = jnp.dot(
        xb, wcat_ref[...], preferred_element_type=jnp.float32
    ).reshape(BN * R, D)
    base_ref[...] = (jnp.dot(xb, root_ref[...], preferred_element_type=jnp.float32)
                     + b_ref[...])


def _tc_b_body(base1_ref, parts_ref, wcat_ref, root_ref, b_ref, y_ref, base2_ref):
    h = jnp.maximum(base1_ref[...] + parts_ref[0] + parts_ref[1], 0.0)
    y_ref[...] = jnp.dot(
        h, wcat_ref[...], preferred_element_type=jnp.float32
    ).reshape(BN * R, D)
    base2_ref[...] = (jnp.dot(h, root_ref[...], preferred_element_type=jnp.float32)
                      + b_ref[...])


def _tc_c_body(base2_ref, parts_ref, out_ref):
    out_ref[...] = base2_ref[...] + parts_ref[0] + parts_ref[1]


def _tc_a(x, wcat, root, b):
    return pl.pallas_call(
        _tc_a_body,
        grid=(N // BN,),
        in_specs=[
            pl.BlockSpec((BN, D), lambda i: (i, 0)),
            pl.BlockSpec((D, R * D), lambda i: (0, 0)),
            pl.BlockSpec((D, D), lambda i: (0, 0)),
            pl.BlockSpec((1, D), lambda i: (0, 0)),
        ],
        out_specs=[
            pl.BlockSpec((BN * R, D), lambda i: (i, 0)),
            pl.BlockSpec((BN, D), lambda i: (i, 0)),
        ],
        out_shape=[
            jax.ShapeDtypeStruct((N * R, D), jnp.float32),
            jax.ShapeDtypeStruct((N, D), jnp.float32),
        ],
    )(x, wcat, root, b)


def _tc_b(base1, parts, wcat, root, b):
    return pl.pallas_call(
        _tc_b_body,
        grid=(N // BN,),
        in_specs=[
            pl.BlockSpec((BN, D), lambda i: (i, 0)),
            pl.BlockSpec((NC, BN, D), lambda i: (0, i, 0)),
            pl.BlockSpec((D, R * D), lambda i: (0, 0)),
            pl.BlockSpec((D, D), lambda i: (0, 0)),
            pl.BlockSpec((1, D), lambda i: (0, 0)),
        ],
        out_specs=[
            pl.BlockSpec((BN * R, D), lambda i: (i, 0)),
            pl.BlockSpec((BN, D), lambda i: (i, 0)),
        ],
        out_shape=[
            jax.ShapeDtypeStruct((N * R, D), jnp.float32),
            jax.ShapeDtypeStruct((N, D), jnp.float32),
        ],
    )(base1, parts, wcat, root, b)


def _tc_c(base2, parts):
    return pl.pallas_call(
        _tc_c_body,
        grid=(N // BN,),
        in_specs=[
            pl.BlockSpec((BN, D), lambda i: (i, 0)),
            pl.BlockSpec((NC, BN, D), lambda i: (0, i, 0)),
        ],
        out_specs=pl.BlockSpec((BN, D), lambda i: (i, 0)),
        out_shape=jax.ShapeDtypeStruct((N, D), jnp.float32),
    )(base2, parts)


def kernel(x, edge_index, edge_type, W1, root1, b1, W2, root2, b2):
    src2 = edge_index[0].reshape(EROWS, B0)
    dst2 = edge_index[1].reshape(EROWS, B0)
    ty2 = edge_type.reshape(EROWS, B0)
    wcat1 = W1.transpose(1, 0, 2).reshape(D, R * D)
    wcat2 = W2.transpose(1, 0, 2).reshape(D, R * D)
    b1r = b1.reshape(1, D)
    b2r = b2.reshape(1, D)

    wtab, sidx2, cidx2 = _phase_p(src2, dst2, ty2)

    y1, base1 = _tc_a(x, wcat1, root1, b1r)
    parts1 = _layer_pass(y1, sidx2, dst2, cidx2, wtab)
    y2, base2 = _tc_b(base1, parts1.reshape(NC, NPAD, D), wcat2, root2, b2r)
    parts2 = _layer_pass(y2, sidx2, dst2, cidx2, wtab)
    return _tc_c(base2, parts2.reshape(NC, NPAD, D))
